# Initial kernel scaffold; baseline (speedup 1.0000x reference)
import functools

import jax
import jax.numpy as jnp
from jax import lax
from jax.experimental import pallas as pl
from jax.experimental.pallas import tpu as pltpu
from jax.experimental.pallas import tpu_sc as plsc

N_NODES = 10000
N_EDGES = 320000
NP = 10240
DUMMY = 10000
D = 128
DEG_W = 16
NG = 128

NTILES = 32
EPT = N_EDGES // NTILES
CHUNK = 128
EPT_PAD = 80 * CHUNK
ROWS_PT = NP // 16

_mesh = plsc.VectorSubcoreMesh(core_axis_name="c", subcore_axis_name="s")


@functools.partial(
    pl.kernel,
    mesh=_mesh,
    out_type=jax.ShapeDtypeStruct((2, NP, DEG_W), jnp.float32),
    scratch_types=[
        pltpu.VMEM((80, CHUNK), jnp.int32),
        pltpu.VMEM((CHUNK, DEG_W), jnp.float32),
        pltpu.VMEM_SHARED((NP, DEG_W), jnp.float32),
    ],
)
def _deg_kernel(dst_hbm, ones_hbm, zeros_hbm, degp_hbm, dst_v, ones_v, dacc):
    c = lax.axis_index("c")
    s = lax.axis_index("s")
    wid = s * 2 + c
    pltpu.sync_copy(dst_hbm.at[wid], dst_v)
    pltpu.sync_copy(ones_hbm, ones_v)
    pltpu.sync_copy(zeros_hbm, dacc.at[pl.ds(s * ROWS_PT, ROWS_PT), :])
    plsc.subcore_barrier()

    def chunk(j, carry):
        pltpu.sync_copy(ones_v, dacc.at[dst_v.at[j]], add=True)
        return carry

    lax.fori_loop(0, 80, chunk, 0)
    plsc.subcore_barrier()
    pltpu.sync_copy(dacc.at[pl.ds(s * ROWS_PT, ROWS_PT), :],
                    degp_hbm.at[c, pl.ds(s * ROWS_PT, ROWS_PT), :])


@functools.partial(
    pl.kernel,
    mesh=_mesh,
    out_type=jax.ShapeDtypeStruct((2, NP, D), jnp.float32),
    scratch_types=[
        pltpu.VMEM((80, CHUNK), jnp.int32),
        pltpu.VMEM((80, CHUNK), jnp.int32),
        pltpu.VMEM((CHUNK, D), jnp.float32),
        pltpu.VMEM((CHUNK, D), jnp.float32),
        pltpu.VMEM_SHARED((NP, D), jnp.float32),
        pltpu.SemaphoreType.DMA,
        pltpu.SemaphoreType.DMA,
    ],
)
def _scatter_kernel(src_hbm, dst_hbm, table_hbm, zeros_hbm, part_hbm,
                    src_v, dst_v, gbuf0, gbuf1, acc, sem0, sem1):
    c = lax.axis_index("c")
    s = lax.axis_index("s")
    wid = s * 2 + c
    pltpu.sync_copy(src_hbm.at[wid], src_v)
    pltpu.sync_copy(dst_hbm.at[wid], dst_v)
    pltpu.sync_copy(zeros_hbm, acc.at[pl.ds(s * ROWS_PT, ROWS_PT), :])
    plsc.subcore_barrier()

    pltpu.async_copy(table_hbm.at[src_v.at[0]], gbuf0, sem0).wait()

    def chunk(j, carry):
        @pl.when(j % 2 == 0)
        def _():
            @pl.when(j + 1 < 80)
            def _():
                pltpu.async_copy(table_hbm.at[src_v.at[j + 1]], gbuf1, sem1).wait()
            pltpu.sync_copy(gbuf0, acc.at[dst_v.at[j]], add=True)

        @pl.when(j % 2 == 1)
        def _():
            @pl.when(j + 1 < 80)
            def _():
                pltpu.async_copy(table_hbm.at[src_v.at[j + 1]], gbuf0, sem0).wait()
            pltpu.sync_copy(gbuf1, acc.at[dst_v.at[j]], add=True)

        return carry

    lax.fori_loop(0, 80, chunk, 0)
    plsc.subcore_barrier()
    pltpu.sync_copy(acc.at[pl.ds(s * ROWS_PT, ROWS_PT), :],
                    part_hbm.at[c, pl.ds(s * ROWS_PT, ROWS_PT), :])


BM = 1250


def _tc_a_body(x_ref, w_ref, d0_ref, d1_ref, hs_ref, dinv_ref):
    deg = d0_ref[:, 0:1] + d1_ref[:, 0:1] + 1.0
    dinv = lax.rsqrt(deg)
    h = jnp.dot(x_ref[...], w_ref[...], preferred_element_type=jnp.float32)
    hs_ref[...] = h * dinv
    dinv_ref[...] = jnp.broadcast_to(dinv, (BM, D))


def _tc_b_body(p0_ref, p1_ref, hs_ref, dinv_ref, b_ref, w_ref, out_ref):
    dinv = dinv_ref[...]
    h = (p0_ref[...] + p1_ref[...] + hs_ref[...]) * dinv + b_ref[...]
    h = jnp.maximum(h, 0.0)
    out_ref[...] = jnp.dot(h, w_ref[...], preferred_element_type=jnp.float32) * dinv


def _tc_c_body(q0_ref, q1_ref, hs_ref, dinv_ref, b_ref, batch_ref, wc_ref,
               bc_ref, out_ref, s_acc, c_acc):
    i = pl.program_id(0)

    @pl.when(i == 0)
    def _():
        s_acc[...] = jnp.zeros_like(s_acc)
        c_acc[...] = jnp.zeros_like(c_acc)

    h = (q0_ref[...] + q1_ref[...] + hs_ref[...]) * dinv_ref[...] + b_ref[...]
    h = jnp.maximum(h, 0.0)
    gids = lax.broadcasted_iota(jnp.int32, (BM, NG), 1)
    oh = (batch_ref[...] == gids).astype(jnp.float32)
    s_acc[...] += lax.dot_general(oh, h, (((0,), (0,)), ((), ())),
                                  preferred_element_type=jnp.float32)
    c_acc[...] += lax.dot_general(oh, jnp.ones((BM, NG), jnp.float32),
                                  (((0,), (0,)), ((), ())),
                                  preferred_element_type=jnp.float32)

    @pl.when(i == pl.num_programs(0) - 1)
    def _():
        pooled = s_acc[...] / jnp.maximum(c_acc[...], 1.0)
        out_ref[...] = jnp.dot(pooled, wc_ref[...],
                               preferred_element_type=jnp.float32) + bc_ref[...]


def _row_spec(shape):
    nd = len(shape)
    return pl.BlockSpec((BM,) + tuple(shape[1:]), lambda i: (i,) + (0,) * (nd - 1))


def _full_spec(shape):
    nd = len(shape)
    return pl.BlockSpec(tuple(shape), lambda i: (0,) * nd)


def _tc_a(x, w1p, d0, d1):
    return pl.pallas_call(
        _tc_a_body,
        grid=(N_NODES // BM,),
        in_specs=[_row_spec((N_NODES, D)), _full_spec((D, D)),
                  _row_spec((N_NODES, DEG_W)), _row_spec((N_NODES, DEG_W))],
        out_specs=[_row_spec((N_NODES, D)), _row_spec((N_NODES, D))],
        out_shape=[jax.ShapeDtypeStruct((N_NODES, D), jnp.float32),
                   jax.ShapeDtypeStruct((N_NODES, D), jnp.float32)],
    )(x, w1p, d0, d1)


def _tc_b(p0, p1, hs, dinvb, b1p, w2p):
    return pl.pallas_call(
        _tc_b_body,
        grid=(N_NODES // BM,),
        in_specs=[_row_spec((N_NODES, D))] * 4 + [_full_spec((1, D)),
                                                  _full_spec((D, D))],
        out_specs=_row_spec((N_NODES, D)),
        out_shape=jax.ShapeDtypeStruct((N_NODES, D), jnp.float32),
    )(p0, p1, hs, dinvb, b1p, w2p)


def _tc_c(q0, q1, hs, dinvb, b2p, batch2d, wcp, bcp):
    return pl.pallas_call(
        _tc_c_body,
        grid=(N_NODES // BM,),
        in_specs=[_row_spec((N_NODES, D))] * 4 + [
            _full_spec((1, D)), _row_spec((N_NODES, 1)),
            _full_spec((D, NG)), _full_spec((1, NG))],
        out_specs=_full_spec((NG, NG)),
        out_shape=jax.ShapeDtypeStruct((NG, NG), jnp.float32),
        scratch_shapes=[pltpu.VMEM((NG, NG), jnp.float32),
                        pltpu.VMEM((NG, NG), jnp.float32)],
    )(q0, q1, hs, dinvb, b2p, batch2d, wcp, bcp)


def kernel(x, edge_index, batch, W1, b1, W2, b2, Wc, bc):
    f32 = jnp.float32
    w1p = jnp.pad(W1.astype(f32), ((0, 0), (0, D - W1.shape[1])))
    w2p = jnp.pad(W2.astype(f32), ((0, D - W2.shape[0]), (0, D - W2.shape[1])))
    wcp = jnp.pad(Wc.astype(f32), ((0, D - Wc.shape[0]), (0, NG - Wc.shape[1])))
    b1p = jnp.pad(b1.astype(f32), (0, D - b1.shape[0])).reshape(1, D)
    b2p = jnp.pad(b2.astype(f32), (0, D - b2.shape[0])).reshape(1, D)
    bcp = jnp.pad(bc.astype(f32), (0, NG - bc.shape[0])).reshape(1, NG)

    src = edge_index[0].reshape(NTILES, EPT)
    dst = edge_index[1].reshape(NTILES, EPT)
    pad = ((0, 0), (0, EPT_PAD - EPT))
    src_r = jnp.pad(src, pad, constant_values=DUMMY).reshape(NTILES, 80, CHUNK)
    dst_r = jnp.pad(dst, pad, constant_values=DUMMY).reshape(NTILES, 80, CHUNK)

    ones_deg = jnp.ones((CHUNK, DEG_W), f32)
    zeros_deg = jnp.zeros((ROWS_PT, DEG_W), f32)
    zeros_scat = jnp.zeros((ROWS_PT, D), f32)

    degp = _deg_kernel(dst_r, ones_deg, zeros_deg)
    d0 = degp[0, :N_NODES, :]
    d1 = degp[1, :N_NODES, :]

    hs1, dinvb = _tc_a(x.astype(f32), w1p, d0, d1)
    t1 = jnp.pad(hs1, ((0, NP - N_NODES), (0, 0)))
    p = _scatter_kernel(src_r, dst_r, t1, zeros_scat)
    hs2 = _tc_b(p[0, :N_NODES, :], p[1, :N_NODES, :], hs1, dinvb, b1p, w2p)

    t2 = jnp.pad(hs2, ((0, NP - N_NODES), (0, 0)))
    q = _scatter_kernel(src_r, dst_r, t2, zeros_scat)
    out = _tc_c(q[0, :N_NODES, :], q[1, :N_NODES, :], hs2, dinvb, b2p,
                batch.reshape(N_NODES, 1), wcp, bcp)
    return out[:, :bc.shape[0]]


# speedup vs baseline: 11.5311x; 11.5311x over previous
import functools

import jax
import jax.numpy as jnp
from jax import lax
from jax.experimental import pallas as pl
from jax.experimental.pallas import tpu as pltpu
from jax.experimental.pallas import tpu_sc as plsc

N_NODES = 10000
N_EDGES = 320000
NP = 10240
DUMMY = 10000
D = 128
DH = 112
DEG_W = 16
NG = 128

NTILES = 32
EPT = N_EDGES // NTILES
CHUNK = 128
EPT_PAD = 80 * CHUNK
ROWS_PT = NP // 16

_mesh = plsc.VectorSubcoreMesh(core_axis_name="c", subcore_axis_name="s")


@functools.partial(
    pl.kernel,
    mesh=_mesh,
    compiler_params=pltpu.CompilerParams(use_tc_tiling_on_sc=False),
    out_type=jax.ShapeDtypeStruct((2, NP, DEG_W), jnp.float32),
    scratch_types=[
        pltpu.VMEM((80, CHUNK), jnp.int32),
        pltpu.VMEM((CHUNK, DEG_W), jnp.float32),
        pltpu.VMEM_SHARED((NP, DEG_W), jnp.float32),
    ],
)
def _deg_kernel(dst_hbm, ones_hbm, zeros_hbm, degp_hbm, dst_v, ones_v, dacc):
    c = lax.axis_index("c")
    s = lax.axis_index("s")
    wid = s * 2 + c
    pltpu.sync_copy(dst_hbm.at[wid], dst_v)
    pltpu.sync_copy(ones_hbm, ones_v)
    pltpu.sync_copy(zeros_hbm, dacc.at[pl.ds(s * ROWS_PT, ROWS_PT), :])
    plsc.subcore_barrier()

    def chunk(j, carry):
        pltpu.sync_copy(ones_v, dacc.at[dst_v.at[j]], add=True)
        return carry

    lax.fori_loop(0, 80, chunk, 0)
    plsc.subcore_barrier()
    pltpu.sync_copy(dacc.at[pl.ds(s * ROWS_PT, ROWS_PT), :],
                    degp_hbm.at[c, pl.ds(s * ROWS_PT, ROWS_PT), :])


@functools.partial(
    pl.kernel,
    mesh=_mesh,
    compiler_params=pltpu.CompilerParams(use_tc_tiling_on_sc=False),
    out_type=jax.ShapeDtypeStruct((2, NP, DH), jnp.float32),
    scratch_types=[
        pltpu.VMEM((80, CHUNK), jnp.int32),
        pltpu.VMEM((80, CHUNK), jnp.int32),
        pltpu.VMEM((CHUNK, DH), jnp.float32),
        pltpu.VMEM((CHUNK, DH), jnp.float32),
        pltpu.VMEM_SHARED((NP, DH), jnp.float32),
        pltpu.SemaphoreType.DMA,
        pltpu.SemaphoreType.DMA,
    ],
)
def _scatter_kernel(src_hbm, dst_hbm, table_hbm, zeros_hbm, part_hbm,
                    src_v, dst_v, gbuf0, gbuf1, acc, sem0, sem1):
    c = lax.axis_index("c")
    s = lax.axis_index("s")
    wid = s * 2 + c
    pltpu.sync_copy(src_hbm.at[wid], src_v)
    pltpu.sync_copy(dst_hbm.at[wid], dst_v)
    pltpu.sync_copy(zeros_hbm, acc.at[pl.ds(s * ROWS_PT, ROWS_PT), :])
    plsc.subcore_barrier()

    pltpu.async_copy(table_hbm.at[src_v.at[0]], gbuf0, sem0).wait()

    def chunk(j, carry):
        @pl.when(j % 2 == 0)
        def _():
            cp = pltpu.async_copy(table_hbm.at[src_v.at[j + 1]], gbuf1, sem1)
            pltpu.sync_copy(gbuf0, acc.at[dst_v.at[j]], add=True)
            cp.wait()

        @pl.when(j % 2 == 1)
        def _():
            cp = pltpu.async_copy(table_hbm.at[src_v.at[j + 1]], gbuf0, sem0)
            pltpu.sync_copy(gbuf1, acc.at[dst_v.at[j]], add=True)
            cp.wait()

        return carry

    lax.fori_loop(0, 79, chunk, 0)
    pltpu.sync_copy(gbuf1, acc.at[dst_v.at[79]], add=True)
    plsc.subcore_barrier()
    pltpu.sync_copy(acc.at[pl.ds(s * ROWS_PT, ROWS_PT), :],
                    part_hbm.at[c, pl.ds(s * ROWS_PT, ROWS_PT), :])


BM = 2000


def _tc_a_body(x_ref, w_ref, d0_ref, d1_ref, hs_ref, dinv_ref):
    deg = d0_ref[:, 0:1] + d1_ref[:, 0:1] + 1.0
    dinv = lax.rsqrt(deg)
    h = jnp.dot(x_ref[...], w_ref[...], preferred_element_type=jnp.float32)
    hs_ref[...] = h * dinv
    dinv_ref[...] = jnp.broadcast_to(dinv, (BM, DH))


def _tc_b_body(p0_ref, p1_ref, hs_ref, dinv_ref, b_ref, w_ref, out_ref):
    dinv = dinv_ref[...]
    h = (p0_ref[...] + p1_ref[...] + hs_ref[...]) * dinv + b_ref[...]
    h = jnp.maximum(h, 0.0)
    out_ref[...] = jnp.dot(h, w_ref[...], preferred_element_type=jnp.float32) * dinv


def _tc_c_body(q0_ref, q1_ref, hs_ref, dinv_ref, b_ref, batch_ref, wc_ref,
               bc_ref, out_ref, s_acc, c_acc):
    i = pl.program_id(0)

    @pl.when(i == 0)
    def _():
        s_acc[...] = jnp.zeros_like(s_acc)
        c_acc[...] = jnp.zeros_like(c_acc)

    h = (q0_ref[...] + q1_ref[...] + hs_ref[...]) * dinv_ref[...] + b_ref[...]
    h = jnp.maximum(h, 0.0)
    gids = lax.broadcasted_iota(jnp.int32, (BM, NG), 1)
    oh = (batch_ref[...] == gids).astype(jnp.float32)
    s_acc[...] += lax.dot_general(oh, h, (((0,), (0,)), ((), ())),
                                  preferred_element_type=jnp.float32)
    c_acc[...] += lax.dot_general(oh, jnp.ones((BM, DH), jnp.float32),
                                  (((0,), (0,)), ((), ())),
                                  preferred_element_type=jnp.float32)

    @pl.when(i == pl.num_programs(0) - 1)
    def _():
        pooled = s_acc[...] / jnp.maximum(c_acc[...], 1.0)
        out_ref[...] = jnp.dot(pooled, wc_ref[...],
                               preferred_element_type=jnp.float32) + bc_ref[...]


def _row_spec(shape):
    nd = len(shape)
    return pl.BlockSpec((BM,) + tuple(shape[1:]), lambda i: (i,) + (0,) * (nd - 1))


def _full_spec(shape):
    nd = len(shape)
    return pl.BlockSpec(tuple(shape), lambda i: (0,) * nd)


def _tc_a(x, w1p, d0, d1):
    return pl.pallas_call(
        _tc_a_body,
        grid=(N_NODES // BM,),
        in_specs=[_row_spec((N_NODES, D)), _full_spec((D, DH)),
                  _row_spec((N_NODES, DEG_W)), _row_spec((N_NODES, DEG_W))],
        out_specs=[_row_spec((N_NODES, DH)), _row_spec((N_NODES, DH))],
        out_shape=[jax.ShapeDtypeStruct((N_NODES, DH), jnp.float32),
                   jax.ShapeDtypeStruct((N_NODES, DH), jnp.float32)],
    )(x, w1p, d0, d1)


def _tc_b(p0, p1, hs, dinvb, b1p, w2p):
    return pl.pallas_call(
        _tc_b_body,
        grid=(N_NODES // BM,),
        in_specs=[_row_spec((N_NODES, DH))] * 4 + [_full_spec((1, DH)),
                                                   _full_spec((DH, DH))],
        out_specs=_row_spec((N_NODES, DH)),
        out_shape=jax.ShapeDtypeStruct((N_NODES, DH), jnp.float32),
    )(p0, p1, hs, dinvb, b1p, w2p)


def _tc_c(q0, q1, hs, dinvb, b2p, batch2d, wcp, bcp):
    return pl.pallas_call(
        _tc_c_body,
        grid=(N_NODES // BM,),
        in_specs=[_row_spec((N_NODES, DH))] * 4 + [
            _full_spec((1, DH)), _row_spec((N_NODES, 1)),
            _full_spec((DH, NG)), _full_spec((1, NG))],
        out_specs=_full_spec((NG, NG)),
        out_shape=jax.ShapeDtypeStruct((NG, NG), jnp.float32),
        scratch_shapes=[pltpu.VMEM((NG, DH), jnp.float32),
                        pltpu.VMEM((NG, DH), jnp.float32)],
    )(q0, q1, hs, dinvb, b2p, batch2d, wcp, bcp)


def kernel(x, edge_index, batch, W1, b1, W2, b2, Wc, bc):
    f32 = jnp.float32
    w1p = jnp.pad(W1.astype(f32), ((0, 0), (0, DH - W1.shape[1])))
    w2p = jnp.pad(W2.astype(f32), ((0, DH - W2.shape[0]), (0, DH - W2.shape[1])))
    wcp = jnp.pad(Wc.astype(f32), ((0, DH - Wc.shape[0]), (0, NG - Wc.shape[1])))
    b1p = jnp.pad(b1.astype(f32), (0, DH - b1.shape[0])).reshape(1, DH)
    b2p = jnp.pad(b2.astype(f32), (0, DH - b2.shape[0])).reshape(1, DH)
    bcp = jnp.pad(bc.astype(f32), (0, NG - bc.shape[0])).reshape(1, NG)

    src = edge_index[0].reshape(NTILES, EPT)
    dst = edge_index[1].reshape(NTILES, EPT)
    pad = ((0, 0), (0, EPT_PAD - EPT))
    src_r = jnp.pad(src, pad, constant_values=DUMMY).reshape(NTILES, 80, CHUNK)
    dst_r = jnp.pad(dst, pad, constant_values=DUMMY).reshape(NTILES, 80, CHUNK)

    ones_deg = jnp.ones((CHUNK, DEG_W), f32)
    zeros_deg = jnp.zeros((ROWS_PT, DEG_W), f32)
    zeros_scat = jnp.zeros((ROWS_PT, DH), f32)

    degp = _deg_kernel(dst_r, ones_deg, zeros_deg)
    d0 = degp[0, :N_NODES, :]
    d1 = degp[1, :N_NODES, :]

    hs1, dinvb = _tc_a(x.astype(f32), w1p, d0, d1)
    t1 = jnp.pad(hs1, ((0, NP - N_NODES), (0, 0)))
    p = _scatter_kernel(src_r, dst_r, t1, zeros_scat)
    hs2 = _tc_b(p[0, :N_NODES, :], p[1, :N_NODES, :], hs1, dinvb, b1p, w2p)

    t2 = jnp.pad(hs2, ((0, NP - N_NODES), (0, 0)))
    q = _scatter_kernel(src_r, dst_r, t2, zeros_scat)
    out = _tc_c(q[0, :N_NODES, :], q[1, :N_NODES, :], hs2, dinvb, b2p,
                batch.reshape(N_NODES, 1), wcp, bcp)
    return out[:, :bc.shape[0]]
